# baseline (speedup 1.0000x reference)
import jax
import jax.numpy as jnp
from jax import lax
from jax.experimental import pallas as pl
from jax.experimental.pallas import tpu as pltpu
from jax.experimental.pallas import tpu_sc as plsc
from jax._src.pallas import mpmd


def kernel(input_ids, table):
    batch_size, seq_len = input_ids.shape
    max_len, d_model = table.shape

    info = plsc.get_sparse_core_info()
    nc, ns = info.num_cores, info.num_subcores
    nw = nc * ns
    scs_batches = 1
    tec_batches = batch_size - scs_batches

    rows_per_w = seq_len // nw
    chunk = 56
    nbuf = 2
    bounds = list(range(0, rows_per_w, chunk)) + [rows_per_w]
    sizes = [bounds[j + 1] - bounds[j] for j in range(len(bounds) - 1)]
    n_chunks = len(sizes)

    rows_per_c = seq_len // nc
    s_chunk = 128
    s_nbuf = 2
    sn_chunks = rows_per_c // s_chunk

    vec_mesh = plsc.VectorSubcoreMesh(core_axis_name="c", subcore_axis_name="s")
    scalar_mesh = plsc.ScalarSubcoreMesh(axis_name="c")

    def tec_fn(table_hbm, out_hbm, bufs, insem, outsem, sbufs, sinsem, soutsem):
        del sbufs, sinsem, soutsem
        wid = lax.axis_index("s") * nc + lax.axis_index("c")
        base = wid * rows_per_w

        def cp_in(i):
            start = base + bounds[i]
            return pltpu.async_copy(
                table_hbm.at[pl.ds(start, sizes[i])],
                bufs.at[i % nbuf, pl.ds(0, sizes[i])],
                insem,
            )

        def cp_out(i, b):
            start = base + bounds[i]
            return pltpu.async_copy(
                bufs.at[i % nbuf, pl.ds(0, sizes[i])],
                out_hbm.at[b, pl.ds(start, sizes[i])],
                outsem,
            )

        h_in = [None] * n_chunks
        h_out = [None] * n_chunks
        h_in[0] = cp_in(0)
        for i in range(n_chunks):
            if i + 1 < n_chunks:
                if i + 1 - nbuf >= 0:
                    for h in h_out[i + 1 - nbuf]:
                        h.wait()
                h_in[i + 1] = cp_in(i + 1)
            h_in[i].wait()
            h_out[i] = [cp_out(i, b) for b in range(tec_batches)]
        for i in range(max(0, n_chunks - nbuf), n_chunks):
            for h in h_out[i]:
                h.wait()

    def scs_fn(table_hbm, out_hbm, bufs, insem, outsem, sbufs, sinsem, soutsem):
        del bufs, insem, outsem
        base = lax.axis_index("c") * rows_per_c

        def cp_in(i):
            start = base + i * s_chunk
            return pltpu.async_copy(
                table_hbm.at[pl.ds(start, s_chunk)], sbufs.at[i % s_nbuf], sinsem
            )

        def cp_out(i, b):
            start = base + i * s_chunk
            return pltpu.async_copy(
                sbufs.at[i % s_nbuf],
                out_hbm.at[tec_batches + b, pl.ds(start, s_chunk)],
                soutsem,
            )

        h_in = [None] * sn_chunks
        h_out = [None] * sn_chunks
        h_in[0] = cp_in(0)
        for i in range(sn_chunks):
            if i + 1 < sn_chunks:
                if i + 1 - s_nbuf >= 0:
                    for h in h_out[i + 1 - s_nbuf]:
                        h.wait()
                h_in[i + 1] = cp_in(i + 1)
            h_in[i].wait()
            h_out[i] = [cp_out(i, b) for b in range(scs_batches)]
        for i in range(max(0, sn_chunks - s_nbuf), sn_chunks):
            for h in h_out[i]:
                h.wait()

    call = mpmd.mpmd_map(
        [(scalar_mesh, scs_fn), (vec_mesh, tec_fn)],
        out_types=jax.ShapeDtypeStruct(
            (batch_size, seq_len, d_model), table.dtype
        ),
        scratch_types=(
            (pltpu.VMEM @ vec_mesh)((nbuf, chunk, d_model), jnp.float32),
            pltpu.SemaphoreType.DMA @ vec_mesh,
            pltpu.SemaphoreType.DMA @ vec_mesh,
            pltpu.VMEM_SHARED((s_nbuf, s_chunk, d_model), jnp.float32),
            pltpu.SemaphoreType.DMA @ scalar_mesh,
            pltpu.SemaphoreType.DMA @ scalar_mesh,
        ),
    )
    return call(table)
